# Initial kernel scaffold; baseline (speedup 1.0000x reference)
import jax
import jax.numpy as jnp
from jax.experimental import pallas as pl


def kernel(x, table):
    raise NotImplementedError("write your pallas kernel here")


# speedup vs baseline: 1.4578x; 1.4578x over previous
import functools

import jax
import jax.numpy as jnp
from jax import lax
from jax.experimental import pallas as pl
from jax.experimental.pallas import tpu as pltpu
from jax.experimental.pallas import tpu_sc as plsc

EMB = 32
IDXW = 128
K = 8
CHUNK = K * IDXW


@functools.lru_cache(maxsize=None)
def _make_gather(n_rows: int, emb: int):
    info = plsc.get_sparse_core_info()
    nw = info.num_cores * info.num_subcores
    rows_per_w = n_rows // nw
    assert rows_per_w * nw == n_rows and rows_per_w % CHUNK == 0
    n_chunks = rows_per_w // CHUNK

    mesh = plsc.VectorSubcoreMesh(core_axis_name="c", subcore_axis_name="s")

    @functools.partial(
        pl.kernel,
        mesh=mesh,
        out_type=jax.ShapeDtypeStruct((n_rows, emb), jnp.float32),
        scratch_types=[
            pltpu.VMEM((K, IDXW), jnp.int32),
            pltpu.VMEM((CHUNK, emb), jnp.float32),
            pltpu.SemaphoreType.DMA,
        ],
        compiler_params=pltpu.CompilerParams(use_tc_tiling_on_sc=False),
    )
    def gather(idx_hbm, table_hbm, out_hbm, idx_v, rows_v, gsem):
        wid = lax.axis_index("s") * info.num_cores + lax.axis_index("c")
        row_base = wid * (rows_per_w // IDXW)

        def body(c, carry):
            irow = row_base + c * K
            pltpu.sync_copy(idx_hbm.at[pl.ds(irow, K)], idx_v)
            handles = []
            for j in range(K):
                handles.append(
                    pltpu.async_copy(
                        table_hbm.at[idx_v.at[j]],
                        rows_v.at[pl.ds(j * IDXW, IDXW)],
                        gsem,
                    )
                )
            for h in handles:
                h.wait()
            pltpu.sync_copy(rows_v, out_hbm.at[pl.ds(irow * IDXW, CHUNK)])
            return carry

        lax.fori_loop(0, n_chunks, body, 0)

    return gather


def kernel(x, table):
    b, h = x.shape
    n_rows = b * h
    idx = jnp.asarray(x, jnp.int32).reshape(n_rows // IDXW, IDXW)
    out = _make_gather(n_rows, table.shape[1])(idx, table)
    return out.reshape(b, h, table.shape[1])


# speedup vs baseline: 1.5011x; 1.0297x over previous
import functools

import jax
import jax.numpy as jnp
from jax import lax
from jax.experimental import pallas as pl
from jax.experimental.pallas import tpu as pltpu
from jax.experimental.pallas import tpu_sc as plsc

EMB = 32
IDXW = 128
K = 4
CHUNK = K * IDXW
NBUF = 5


@functools.lru_cache(maxsize=None)
def _make_gather(n_rows: int, emb: int):
    info = plsc.get_sparse_core_info()
    nw = info.num_cores * info.num_subcores
    rows_per_w = n_rows // nw
    assert rows_per_w * nw == n_rows and rows_per_w % (CHUNK * NBUF) == 0
    n_chunks = rows_per_w // CHUNK
    n_outer = n_chunks // NBUF

    mesh = plsc.VectorSubcoreMesh(core_axis_name="c", subcore_axis_name="s")

    @functools.partial(
        pl.kernel,
        mesh=mesh,
        out_type=jax.ShapeDtypeStruct((n_rows, emb), jnp.float32),
        scratch_types=[
            pltpu.VMEM((NBUF, K, IDXW), jnp.int32),
            pltpu.VMEM((NBUF, CHUNK, emb), jnp.float32),
            pltpu.SemaphoreType.DMA((NBUF,)),
            pltpu.SemaphoreType.DMA((NBUF,)),
            pltpu.SemaphoreType.DMA((NBUF,)),
        ],
        compiler_params=pltpu.CompilerParams(use_tc_tiling_on_sc=False),
    )
    def gather(idx_hbm, table_hbm, out_hbm, idx_v, rows_v, isem, gsem, osem):
        wid = lax.axis_index("s") * info.num_cores + lax.axis_index("c")
        base_k = wid * n_chunks * K

        def idx_cp(c, b):
            return pltpu.make_async_copy(
                idx_hbm.at[pl.ds(base_k + c * K, K)], idx_v.at[b], isem.at[b])

        def gath_cp(b, j):
            return pltpu.make_async_copy(
                table_hbm.at[idx_v.at[b, j]],
                rows_v.at[b, pl.ds(j * IDXW, IDXW)],
                gsem.at[b])

        def out_cp(c, b):
            return pltpu.make_async_copy(
                rows_v.at[b],
                out_hbm.at[pl.ds((base_k + c * K) * IDXW, CHUNK)],
                osem.at[b])

        for b in range(NBUF):
            idx_cp(b, b).start()

        def outer(o, carry):
            cc = o * NBUF
            for b in range(NBUF):
                c = cc + b
                pb = (b - 1) % NBUF

                @pl.when(o > 0)
                def _():
                    out_cp(c - NBUF, b).wait()

                idx_cp(c, b).wait()
                for j in range(K):
                    gath_cp(b, j).start()

                @pl.when(c > 0)
                def _():
                    for j in range(K):
                        gath_cp(pb, j).wait()
                    out_cp(c - 1, pb).start()

                    @pl.when(c - 1 + NBUF < n_chunks)
                    def _():
                        idx_cp(c - 1 + NBUF, pb).start()

            return carry

        lax.fori_loop(0, n_outer, outer, 0)

        last = n_chunks - 1
        lb = last % NBUF
        for j in range(K):
            gath_cp(lb, j).wait()
        out_cp(last, lb).start()
        for b in range(NBUF):
            out_cp(n_chunks - NBUF + b, b).wait()

    return gather


def kernel(x, table):
    b, h = x.shape
    n_rows = b * h
    idx = jnp.asarray(x, jnp.int32).reshape(n_rows // IDXW, IDXW)
    out = _make_gather(n_rows, table.shape[1])(idx, table)
    return out.reshape(b, h, table.shape[1])
